# baseline (speedup 1.0000x reference)
import functools

import jax
import jax.numpy as jnp
from jax import lax
from jax.experimental import pallas as pl
from jax.experimental.pallas import tpu as pltpu
from jax.experimental.pallas import tpu_sc as plsc

_NUM_WORKERS = 32
_CHUNK = 512


@functools.partial(jax.jit, static_argnums=())
def _embed(ids_flat, table):
    b_total = ids_flat.shape[0]
    d = table.shape[1]
    b_per_w = b_total // _NUM_WORKERS
    n_chunks = b_per_w // _CHUNK

    mesh = plsc.VectorSubcoreMesh(core_axis_name="c", subcore_axis_name="s")

    @functools.partial(
        pl.kernel,
        mesh=mesh,
        out_type=jax.ShapeDtypeStruct((b_total, d), jnp.float32),
        scratch_types=[
            pltpu.VMEM((_CHUNK,), jnp.int32),
            pltpu.VMEM((_CHUNK, d), jnp.float32),
            pltpu.SemaphoreType.DMA,
        ],
        compiler_params=pltpu.CompilerParams(use_tc_tiling_on_sc=False),
    )
    def k(ids_hbm, table_hbm, out_hbm, idx_v, rows_v, sem):
        wid = lax.axis_index("s") * 2 + lax.axis_index("c")
        base = wid * b_per_w

        def body(g, carry):
            off = base + g * _CHUNK
            pltpu.sync_copy(ids_hbm.at[pl.ds(off, _CHUNK)], idx_v)
            pltpu.async_copy(table_hbm.at[idx_v], rows_v, sem).wait()
            pltpu.sync_copy(rows_v, out_hbm.at[pl.ds(off, _CHUNK), :])
            return carry

        lax.fori_loop(0, n_chunks, body, 0)

    return k(ids_flat, table)


def kernel(ids, table):
    b, s = ids.shape
    d = table.shape[1]
    out = _embed(ids.reshape(b * s).astype(jnp.int32), table)
    return out.reshape(b, s, d)


# speedup vs baseline: 1.0456x; 1.0456x over previous
import functools

import jax
import jax.numpy as jnp
from jax import lax
from jax.experimental import pallas as pl
from jax.experimental.pallas import tpu as pltpu
from jax.experimental.pallas import tpu_sc as plsc

_NUM_WORKERS = 32
_CHUNK = 512


@jax.jit
def _embed(ids_flat, table):
    b_total = ids_flat.shape[0]
    d = table.shape[1]
    b_per_w = b_total // _NUM_WORKERS
    n_chunks = b_per_w // _CHUNK
    n2 = n_chunks // 2

    mesh = plsc.VectorSubcoreMesh(core_axis_name="c", subcore_axis_name="s")

    @functools.partial(
        pl.kernel,
        mesh=mesh,
        out_type=jax.ShapeDtypeStruct((b_total, d), jnp.float32),
        scratch_types=[
            pltpu.VMEM((2, _CHUNK), jnp.int32),
            pltpu.VMEM((2, _CHUNK, d), jnp.float32),
            pltpu.SemaphoreType.DMA,
            pltpu.SemaphoreType.DMA,
            pltpu.SemaphoreType.DMA,
            pltpu.SemaphoreType.DMA,
            pltpu.SemaphoreType.DMA,
            pltpu.SemaphoreType.DMA,
        ],
        compiler_params=pltpu.CompilerParams(use_tc_tiling_on_sc=False),
    )
    def k(ids_hbm, table_hbm, out_hbm, idx_v, rows_v, si0, si1, sg0, sg1,
          so0, so1):
        sem_i = (si0, si1)
        sem_g = (sg0, sg1)
        sem_o = (so0, so1)
        wid = lax.axis_index("s") * 2 + lax.axis_index("c")
        base = wid * b_per_w

        def idx_start(g, b):
            off = base + g * _CHUNK
            pltpu.async_copy(ids_hbm.at[pl.ds(off, _CHUNK)], idx_v.at[b],
                             sem_i[b])

        def idx_wait(g, b):
            off = base + g * _CHUNK
            pltpu.make_async_copy(ids_hbm.at[pl.ds(off, _CHUNK)],
                                  idx_v.at[b], sem_i[b]).wait()

        def out_wait(g, b):
            off = base + g * _CHUNK
            pltpu.make_async_copy(rows_v.at[b],
                                  out_hbm.at[pl.ds(off, _CHUNK), :],
                                  sem_o[b]).wait()

        for b in range(2):
            idx_start(b, b)
        for b in range(2):
            idx_wait(b, b)
            pltpu.async_copy(table_hbm.at[idx_v.at[b]], rows_v.at[b],
                             sem_g[b]).wait()
            idx_start(b + 2, b)
            off = base + b * _CHUNK
            pltpu.async_copy(rows_v.at[b], out_hbm.at[pl.ds(off, _CHUNK), :],
                             sem_o[b])

        def body(t, carry):
            for b in range(2):
                g = t * 2 + b
                off = base + g * _CHUNK
                idx_wait(g, b)
                out_wait(g - 2, b)
                pltpu.async_copy(table_hbm.at[idx_v.at[b]], rows_v.at[b],
                                 sem_g[b]).wait()
                g2 = jnp.minimum(g + 2, n_chunks - 1)
                idx_start(g2, b)
                pltpu.async_copy(rows_v.at[b],
                                 out_hbm.at[pl.ds(off, _CHUNK), :], sem_o[b])
            return carry

        lax.fori_loop(1, n2, body, 0)

        for b in range(2):
            idx_wait(0, b)
            out_wait(n_chunks - 2 + b, b)

    return k(ids_flat, table)


def kernel(ids, table):
    b, s = ids.shape
    d = table.shape[1]
    out = _embed(ids.reshape(b * s).astype(jnp.int32), table)
    return out.reshape(b, s, d)
